# baseline (speedup 1.0000x reference)
import functools

import jax
import jax.numpy as jnp
from jax.experimental import pallas as pl
from jax.experimental.pallas import tpu as pltpu

_KEEP_RATIO = 0.7


def _ohem_kernel(targets_ref, x_ref, out_ref, ce_ref, *, nblocks, block_rows, rank):
    i = pl.program_id(0)
    x = x_ref[...]
    rows, ncls = x.shape

    m = jnp.max(x, axis=1, keepdims=True)
    s = jnp.sum(jnp.exp(x - m), axis=1)
    lse = m[:, 0] + jnp.log(s)

    t = targets_ref[0, 0, :]
    cls_iota = jax.lax.broadcasted_iota(jnp.int32, (rows, ncls), 1)
    tl = jnp.sum(jnp.where(cls_iota == t[:, None], x, 0.0), axis=1)

    ce = (lse - tl) + 0.0
    ce_ref[i, :] = ce

    @pl.when(i == nblocks - 1)
    def _select():
        int_min = jnp.int32(-2147483648)
        ce_all = ce_ref[...]
        u = jax.lax.bitcast_convert_type(ce_all, jnp.int32)
        keys = jnp.where(u < 0, u ^ jnp.int32(0x7FFFFFFF), u)

        p = jnp.int32(0)
        for sh in range(28, -1, -4):
            js = jnp.int32(0)
            for j in range(1, 16):
                jv = (j << sh) & 0xFFFFFFFF
                jv = jv - (1 << 32) if jv >= (1 << 31) else jv
                cand_cmp = (p | jnp.int32(jv)) ^ int_min
                cnt = jnp.sum((keys < cand_cmp).astype(jnp.int32))
                js += (cnt <= rank).astype(jnp.int32)
            p = p | jax.lax.shift_left(js, jnp.int32(sh))
        thr_key = p ^ int_min

        mask = (keys >= thr_key).astype(jnp.float32)
        kept_sum = jnp.sum(ce_all * mask)
        kept_cnt = jnp.sum(mask)
        out_ref[0, 0] = kept_sum / (kept_cnt + 1e-8)


def kernel(inputs, targets):
    batch, ncls = inputs.shape
    block_rows = 4096
    nblocks = batch // block_rows
    num_keep = int(batch * _KEEP_RATIO)
    rank = batch - num_keep

    targets3 = targets.astype(jnp.int32).reshape(nblocks, 1, block_rows)

    out = pl.pallas_call(
        functools.partial(
            _ohem_kernel, nblocks=nblocks, block_rows=block_rows, rank=rank
        ),
        grid=(nblocks,),
        in_specs=[
            pl.BlockSpec((1, 1, block_rows), lambda i: (i, 0, 0)),
            pl.BlockSpec((block_rows, ncls), lambda i: (i, 0)),
        ],
        out_specs=pl.BlockSpec(memory_space=pltpu.SMEM),
        out_shape=jax.ShapeDtypeStruct((1, 1), jnp.float32),
        scratch_shapes=[pltpu.VMEM((nblocks, block_rows), jnp.float32)],
    )(targets3, inputs)
    return out[0, 0]


# speedup vs baseline: 1.0473x; 1.0473x over previous
import functools

import jax
import jax.numpy as jnp
from jax.experimental import pallas as pl
from jax.experimental.pallas import tpu as pltpu

_KEEP_RATIO = 0.7


def _ohem_kernel(targets_ref, x_ref, out_ref, ce_ref, *, nblocks, block_rows, rank):
    i = pl.program_id(0)
    x = x_ref[...]
    rows, ncls = x.shape

    t = targets_ref[0, 0, :]
    cls_iota = jax.lax.broadcasted_iota(jnp.int32, (rows, ncls), 1)
    m = jnp.max(x, axis=1, keepdims=True)
    s = jnp.sum(jnp.exp(x - m), axis=1)
    lse = m[:, 0] + jnp.log(s)
    tl = jnp.sum(jnp.where(cls_iota == t[:, None], x, 0.0), axis=1)
    ce = (lse - tl) + 0.0
    ce_ref[i, :] = ce

    @pl.when(i == nblocks - 1)
    def _select():
        int_min = jnp.int32(-2147483648)
        ce_all = ce_ref[...]
        u = jax.lax.bitcast_convert_type(ce_all, jnp.int32)
        keys = jnp.where(u < 0, u ^ jnp.int32(0x7FFFFFFF), u)

        p = jnp.int32(0)
        for sh in range(28, -1, -4):
            js = jnp.int32(0)
            for j in range(1, 16):
                jv = (j << sh) & 0xFFFFFFFF
                jv = jv - (1 << 32) if jv >= (1 << 31) else jv
                cand_cmp = (p | jnp.int32(jv)) ^ int_min
                cnt = jnp.sum((keys < cand_cmp).astype(jnp.int32))
                js += (cnt <= rank).astype(jnp.int32)
            p = p | jax.lax.shift_left(js, jnp.int32(sh))
        thr_key = p ^ int_min

        mask = (keys >= thr_key).astype(jnp.float32)
        kept_sum = jnp.sum(ce_all * mask)
        kept_cnt = jnp.sum(mask)
        out_ref[0, 0] = kept_sum / (kept_cnt + 1e-8)


def kernel(inputs, targets):
    batch, ncls = inputs.shape
    block_rows = 2048
    nblocks = batch // block_rows
    num_keep = int(batch * _KEEP_RATIO)
    rank = batch - num_keep

    targets3 = targets.astype(jnp.int32).reshape(nblocks, 1, block_rows)

    out = pl.pallas_call(
        functools.partial(
            _ohem_kernel, nblocks=nblocks, block_rows=block_rows, rank=rank
        ),
        grid=(nblocks,),
        in_specs=[
            pl.BlockSpec((1, 1, block_rows), lambda i: (i, 0, 0)),
            pl.BlockSpec((block_rows, ncls), lambda i: (i, 0)),
        ],
        out_specs=pl.BlockSpec(memory_space=pltpu.SMEM),
        out_shape=jax.ShapeDtypeStruct((1, 1), jnp.float32),
        scratch_shapes=[pltpu.VMEM((nblocks, block_rows), jnp.float32)],
    )(targets3, inputs)
    return out[0, 0]
